# baseline (speedup 1.0000x reference)
import functools

import jax
import jax.numpy as jnp
from jax import lax
from jax.experimental import pallas as pl
from jax.experimental.pallas import tpu as pltpu
from jax.experimental.pallas import tpu_sc as plsc

N = 10000
NPAD = 10240
E = 320000
CHUNK = 64
CPT = 160
EPAD = 32 * CPT * CHUNK

_mesh = lambda: plsc.VectorSubcoreMesh(core_axis_name="c", subcore_axis_name="s")


def _sc_deg(dst2d, zeros128, ones128):
    @functools.partial(
        pl.kernel,
        mesh=_mesh(),
        out_type=jax.ShapeDtypeStruct((2, NPAD, 128), jnp.float32),
        scratch_types=[
            pltpu.VMEM((CPT, CHUNK), jnp.int32),
            pltpu.VMEM((CHUNK, 128), jnp.float32),
            pltpu.VMEM((CHUNK, 128), jnp.float32),
            pltpu.VMEM_SHARED((NPAD, 128), jnp.float32),
        ],
    )
    def k(dst_hbm, z_hbm, o_hbm, out, dstv, zbuf, onesbuf, acc):
        c = lax.axis_index("c")
        s = lax.axis_index("s")
        wid = c * 16 + s
        pltpu.sync_copy(dst_hbm.at[pl.ds(wid * CPT, CPT)], dstv)
        pltpu.sync_copy(z_hbm, zbuf)
        pltpu.sync_copy(o_hbm, onesbuf)
        for j in range(10):
            pltpu.sync_copy(zbuf, acc.at[pl.ds(s * 640 + j * CHUNK, CHUNK)])
        plsc.subcore_barrier()

        def body(g, _):
            pltpu.sync_copy(onesbuf, acc.at[dstv.at[g]], add=True)
            return 0

        lax.fori_loop(0, CPT, body, 0)
        plsc.subcore_barrier()
        pltpu.sync_copy(acc.at[pl.ds(s * 640, 640)], out.at[c, pl.ds(s * 640, 640)])

    return k(dst2d, zeros128, ones128)


def _sc_agg(tables, src2d, dst2d, zeros128):
    T = len(tables)

    NB = 4

    @functools.partial(
        pl.kernel,
        mesh=_mesh(),
        out_type=[jax.ShapeDtypeStruct((2, NPAD, 128), jnp.float32) for _ in range(T)],
        scratch_types=[
            pltpu.VMEM((CPT // 4, CHUNK), jnp.int32),
            pltpu.VMEM((CPT // 4, CHUNK), jnp.int32),
        ] + [pltpu.VMEM((CHUNK, 128), jnp.float32) for _ in range(NB)] + [
            pltpu.VMEM_SHARED((NPAD, 128), jnp.float32),
        ] + [pltpu.SemaphoreType.DMA for _ in range(2 * NB)],
    )
    def k(*refs):
        us = refs[:T]
        src_hbm, dst_hbm, z_hbm = refs[T:T + 3]
        outs = refs[T + 3:2 * T + 3]
        rest = refs[2 * T + 3:]
        srcv, dstv = rest[0], rest[1]
        rows = rest[2:2 + NB]
        acc = rest[2 + NB]
        gs = rest[3 + NB:3 + 2 * NB]
        ss = rest[3 + 2 * NB:3 + 3 * NB]
        c = lax.axis_index("c")
        s = lax.axis_index("s")
        wid = c * 16 + s
        hcpt = CPT // 4

        for t in range(T):
            u = us[t]
            pltpu.sync_copy(z_hbm, rows[0])
            for j in range(10):
                pltpu.sync_copy(rows[0], acc.at[pl.ds(s * 640 + j * CHUNK, CHUNK)])
            plsc.subcore_barrier()

            for half in range(4):
                base = wid * CPT + half * hcpt
                pltpu.sync_copy(src_hbm.at[pl.ds(base, hcpt)], srcv)
                pltpu.sync_copy(dst_hbm.at[pl.ds(base, hcpt)], dstv)

                for b in range(NB):
                    pltpu.make_async_copy(u.at[srcv.at[b]], rows[b], gs[b]).start()

                def body(i, _):
                    for b in range(NB):
                        g = NB * i + b
                        pltpu.make_async_copy(u.at[srcv.at[g]], rows[b], gs[b]).wait()
                        pltpu.make_async_copy(rows[b], acc.at[dstv.at[g]],
                                              ss[b]).start(add=True)
                    for b in range(NB):
                        g = NB * i + b
                        pltpu.make_async_copy(rows[b], acc.at[dstv.at[g]],
                                              ss[b]).wait()
                        nxt = g + NB

                        @pl.when(nxt < hcpt)
                        def _():
                            pltpu.make_async_copy(
                                u.at[srcv.at[nxt]], rows[b], gs[b]).start()

                    return 0

                lax.fori_loop(0, hcpt // NB, body, 0)
            plsc.subcore_barrier()
            pltpu.sync_copy(acc.at[pl.ds(s * 640, 640)],
                            outs[t].at[c, pl.ds(s * 640, 640)])
            if t + 1 < T:
                plsc.subcore_barrier()

    out = k(*tables, src2d, dst2d, zeros128)
    return list(out) if isinstance(out, (list, tuple)) else [out]


def _dot(a, b):
    return jnp.dot(a, b, preferred_element_type=jnp.float32)


def _k_dinv(degp):
    def body(d_ref, o_ref):
        s = d_ref[0, :, 0:1] + d_ref[1, :, 0:1] + 1.0
        o_ref[...] = 1.0 / jnp.sqrt(s)

    return pl.pallas_call(
        body,
        grid=(10,),
        in_specs=[pl.BlockSpec((2, 1000, 128), lambda g: (0, g, 0))],
        out_specs=pl.BlockSpec((1000, 1), lambda g: (g, 0)),
        out_shape=jax.ShapeDtypeStruct((N, 1), jnp.float32),
    )(degp)


def _k_embed(ev, cs, tr, env, act, dinv, p1, p2):
    def body(ev_ref, cs_ref, tr_ref, env_ref, act_ref, dinv_ref,
             wev1, bev1, wcs1, bcs1, wtr1, btr1, wenv1, benv1, wg0x1, wg0a1,
             wev2, bev2, wcs2, bcs2, wtr2, btr2, wenv2, benv2, wg0x2, wg0a2,
             u1_ref, u2_ref):
        g = pl.program_id(0)
        a = act_ref[...]
        di = dinv_ref[...]
        branches = [
            (0, 4, ev_ref, wev1, bev1, wev2, bev2),
            (4, 7, cs_ref, wcs1, bcs1, wcs2, bcs2),
            (7, 9, tr_ref, wtr1, btr1, wtr2, btr2),
            (9, 10, env_ref, wenv1, benv1, wenv2, benv2),
        ]
        for lo, hi, src_ref, wA, bA, wB, bB in branches:
            @pl.when((g >= lo) & (g < hi))
            def _(lo=lo, src_ref=src_ref, wA=wA, bA=bA, wB=wB, bB=bB):
                blk = src_ref[pl.ds((g - lo) * 1000, 1000), :]
                x1 = jax.nn.relu(_dot(blk, wA[...]) + bA[...])
                x2 = jax.nn.relu(_dot(blk, wB[...]) + bB[...])
                u1_ref[...] = di * (_dot(x1, wg0x1[...]) + a * wg0a1[...])
                u2_ref[...] = di * (_dot(x2, wg0x2[...]) + a * wg0a2[...])

    full = lambda shape: pl.BlockSpec(shape, lambda g: tuple(0 for _ in shape))
    rowblk = pl.BlockSpec((1000, 1), lambda g: (g, 0))
    w1 = [full(p1[k].shape) for k in ("W_ev", "b_ev", "W_cs", "b_cs", "W_tr",
                                      "b_tr", "W_env", "b_env", "W_g0x", "W_g0a")]
    w2 = [full(p2[k].shape) for k in ("W_ev", "b_ev", "W_cs", "b_cs", "W_tr",
                                      "b_tr", "W_env", "b_env", "W_g0x", "W_g0a")]
    args1 = [p1[k] for k in ("W_ev", "b_ev", "W_cs", "b_cs", "W_tr", "b_tr",
                             "W_env", "b_env", "W_g0x", "W_g0a")]
    args2 = [p2[k] for k in ("W_ev", "b_ev", "W_cs", "b_cs", "W_tr", "b_tr",
                             "W_env", "b_env", "W_g0x", "W_g0a")]
    return pl.pallas_call(
        body,
        grid=(10,),
        in_specs=[full(ev.shape), full(cs.shape), full(tr.shape), full(env.shape),
                  rowblk, rowblk] + w1 + w2,
        out_specs=[pl.BlockSpec((1000, 128), lambda g: (g, 0))] * 2,
        out_shape=[jax.ShapeDtypeStruct((N, 128), jnp.float32)] * 2,
    )(ev, cs, tr, env, act, dinv, *args1, *args2)


def _k_mid(r0_1, r0_2, u0_1, u0_2, dinv, bg0_1, bg0_2):
    def body(r1_ref, r2_ref, u1_ref, u2_ref, dinv_ref, b1_ref, b2_ref,
             o1_ref, o2_ref):
        di = dinv_ref[...]
        h1 = jax.nn.relu(di * (r1_ref[0] + r1_ref[1] + u1_ref[...]) + b1_ref[...])
        h2 = jax.nn.relu(di * (r2_ref[0] + r2_ref[1] + u2_ref[...]) + b2_ref[...])
        o1_ref[...] = di * h1
        o2_ref[...] = di * h2

    rblk = pl.BlockSpec((2, 1000, 128), lambda g: (0, g, 0))
    ublk = pl.BlockSpec((1000, 128), lambda g: (g, 0))
    bblk = pl.BlockSpec((1, 128), lambda g: (0, 0))
    return pl.pallas_call(
        body,
        grid=(10,),
        in_specs=[rblk, rblk, ublk, ublk, pl.BlockSpec((1000, 1), lambda g: (g, 0)),
                  bblk, bblk],
        out_specs=[ublk, ublk],
        out_shape=[jax.ShapeDtypeStruct((N, 128), jnp.float32)] * 2,
    )(r0_1, r0_2, u0_1, u0_2, dinv, bg0_1, bg0_2)


def _k_l2(r1_1, r1_2, u1_1, u1_2, dinv, wg1_1, bg1_1, wg1_2, bg1_2):
    def body(r1_ref, r2_ref, u1_ref, u2_ref, dinv_ref, w1_ref, b1_ref,
             w2_ref, b2_ref, o1a, o1b, o2a, o2b):
        di = dinv_ref[...]
        agg1 = di * (r1_ref[0] + r1_ref[1] + u1_ref[...])
        agg2 = di * (r2_ref[0] + r2_ref[1] + u2_ref[...])
        h1 = di * jax.nn.relu(_dot(agg1, w1_ref[...]) + b1_ref[...])
        h2 = di * jax.nn.relu(_dot(agg2, w2_ref[...]) + b2_ref[...])
        o1a[...] = h1[:, :128]
        o1b[...] = h1[:, 128:]
        o2a[...] = h2[:, :128]
        o2b[...] = h2[:, 128:]

    rblk = pl.BlockSpec((2, 1000, 128), lambda g: (0, g, 0))
    ublk = pl.BlockSpec((1000, 128), lambda g: (g, 0))
    return pl.pallas_call(
        body,
        grid=(10,),
        in_specs=[rblk, rblk, ublk, ublk, pl.BlockSpec((1000, 1), lambda g: (g, 0)),
                  pl.BlockSpec((128, 256), lambda g: (0, 0)),
                  pl.BlockSpec((1, 256), lambda g: (0, 0)),
                  pl.BlockSpec((128, 256), lambda g: (0, 0)),
                  pl.BlockSpec((1, 256), lambda g: (0, 0))],
        out_specs=[ublk] * 4,
        out_shape=[jax.ShapeDtypeStruct((N, 128), jnp.float32)] * 4,
    )(r1_1, r1_2, u1_1, u1_2, dinv, wg1_1, bg1_1, wg1_2, bg1_2)


def _k_final(r2s, u2s, dinv, p1, p2):
    def body(r1a, r1b, r2a, r2b, u1a, u1b, u2a, u2b, dinv_ref,
             wg2_1, bg2_1, w1_1, b1_1, w2_1, b2_1, w3_1, b3_1,
             wg2_2, bg2_2, w1_2, b1_2, w2_2, b2_2, w3_2, b3_2,
             q1_ref, q2_ref):
        di = dinv_ref[...]

        def critic(ra, rb, ua, ub, wg2, bg2, w1, b1, w2, b2, w3, b3, q_ref):
            agga = di * (ra[0] + ra[1] + ua[...])
            aggb = di * (rb[0] + rb[1] + ub[...])
            h2 = jax.nn.relu(_dot(agga, wg2[pl.ds(0, 128), :])
                             + _dot(aggb, wg2[pl.ds(128, 128), :]) + bg2[...])
            pooled = jnp.sum(h2, axis=0, keepdims=True) * (1.0 / 1000.0)
            y = jax.nn.relu(_dot(pooled, w1[...]) + b1[...])
            y = jax.nn.relu(_dot(y, w2[...]) + b2[...])
            q = _dot(y, w3[...]) + b3[...]
            q_ref[...] = jnp.broadcast_to(q, (1, 1, 128))

        critic(r1a, r1b, u1a, u1b, wg2_1, bg2_1, w1_1, b1_1, w2_1, b2_1,
               w3_1, b3_1, q1_ref)
        critic(r2a, r2b, u2a, u2b, wg2_2, bg2_2, w1_2, b1_2, w2_2, b2_2,
               w3_2, b3_2, q2_ref)

    rblk = pl.BlockSpec((2, 1000, 128), lambda g: (0, g, 0))
    ublk = pl.BlockSpec((1000, 128), lambda g: (g, 0))
    full = lambda shape: pl.BlockSpec(shape, lambda g: tuple(0 for _ in shape))
    wkeys = ("W_g2", "b_g2", "W1", "b1", "W2", "b2", "W3", "b3")
    wspecs = [full(p1[k].shape) for k in wkeys] + [full(p2[k].shape) for k in wkeys]
    wargs = [p1[k] for k in wkeys] + [p2[k] for k in wkeys]
    q1, q2 = pl.pallas_call(
        body,
        grid=(10,),
        in_specs=[rblk] * 4 + [ublk] * 4
                 + [pl.BlockSpec((1000, 1), lambda g: (g, 0))] + wspecs,
        out_specs=[pl.BlockSpec((1, 1, 128), lambda g: (g, 0, 0))] * 2,
        out_shape=[jax.ShapeDtypeStruct((10, 1, 128), jnp.float32)] * 2,
    )(*r2s, *u2s, dinv, *wargs)
    return q1[:, 0, :1], q2[:, 0, :1]


def _prep_params(p):
    q = dict(p)
    for k in ("b_ev", "b_cs", "b_tr", "b_env", "b_g0", "b_g1", "b_g2",
              "b1", "b2", "b3"):
        q[k] = p[k].reshape(1, -1)
    q["W_g0x"] = p["W_g0"][:128]
    q["W_g0a"] = p["W_g0"][128].reshape(1, -1)
    return q


def kernel(ev_features, cs_features, tr_features, env_features, edge_index,
           ev_indexes, cs_indexes, tr_indexes, env_indexes, action, batch,
           params1, params2):
    src = edge_index[0]
    dst = edge_index[1]
    npad = EPAD - E
    src2d = jnp.concatenate([src, jnp.zeros((npad,), jnp.int32)]).reshape(-1, CHUNK)
    dst2d = jnp.concatenate([dst, jnp.full((npad,), N, jnp.int32)]).reshape(-1, CHUNK)
    zeros128 = jnp.zeros((CHUNK, 128), jnp.float32)
    ones128 = jnp.ones((CHUNK, 128), jnp.float32)
    act = action.reshape(-1, 1)
    p1 = _prep_params(params1)
    p2 = _prep_params(params2)

    degp = _sc_deg(dst2d, zeros128, ones128)
    dinv = _k_dinv(degp)
    u0_1, u0_2 = _k_embed(ev_features, cs_features, tr_features, env_features,
                          act, dinv, p1, p2)
    r0_1, r0_2 = _sc_agg([u0_1, u0_2], src2d, dst2d, zeros128)
    u1_1, u1_2 = _k_mid(r0_1, r0_2, u0_1, u0_2, dinv, p1["b_g0"], p2["b_g0"])
    r1_1, r1_2 = _sc_agg([u1_1, u1_2], src2d, dst2d, zeros128)
    u2 = _k_l2(r1_1, r1_2, u1_1, u1_2, dinv, p1["W_g1"], p1["b_g1"],
               p2["W_g1"], p2["b_g1"])
    r2 = _sc_agg(list(u2), src2d, dst2d, zeros128)
    return _k_final(r2, u2, dinv, p1, p2)


# speedup vs baseline: 1.1208x; 1.1208x over previous
import functools

import jax
import jax.numpy as jnp
from jax import lax
from jax.experimental import pallas as pl
from jax.experimental.pallas import tpu as pltpu
from jax.experimental.pallas import tpu_sc as plsc

N = 10000
NPAD = 10240
E = 320000
CHUNK = 128
CPT = 80
EPAD = 32 * CPT * CHUNK

_mesh = lambda: plsc.VectorSubcoreMesh(core_axis_name="c", subcore_axis_name="s")


def _sc_deg(dst2d, zeros128, ones128):
    @functools.partial(
        pl.kernel,
        mesh=_mesh(),
        out_type=jax.ShapeDtypeStruct((2, NPAD, 128), jnp.float32),
        scratch_types=[
            pltpu.VMEM((CPT, CHUNK), jnp.int32),
            pltpu.VMEM((CHUNK, 128), jnp.float32),
            pltpu.VMEM((CHUNK, 128), jnp.float32),
            pltpu.VMEM_SHARED((NPAD, 128), jnp.float32),
        ],
    )
    def k(dst_hbm, z_hbm, o_hbm, out, dstv, zbuf, onesbuf, acc):
        c = lax.axis_index("c")
        s = lax.axis_index("s")
        wid = c * 16 + s
        pltpu.sync_copy(dst_hbm.at[pl.ds(wid * CPT, CPT)], dstv)
        pltpu.sync_copy(z_hbm, zbuf)
        pltpu.sync_copy(o_hbm, onesbuf)
        for j in range(640 // CHUNK):
            pltpu.sync_copy(zbuf, acc.at[pl.ds(s * 640 + j * CHUNK, CHUNK)])
        plsc.subcore_barrier()

        def body(g, _):
            pltpu.sync_copy(onesbuf, acc.at[dstv.at[g]], add=True)
            return 0

        lax.fori_loop(0, CPT, body, 0)
        plsc.subcore_barrier()
        pltpu.sync_copy(acc.at[pl.ds(s * 640, 640)], out.at[c, pl.ds(s * 640, 640)])

    return k(dst2d, zeros128, ones128)


def _sc_agg(tables, src2d, dst2d, zeros128):
    T = len(tables)

    @functools.partial(
        pl.kernel,
        mesh=_mesh(),
        out_type=[jax.ShapeDtypeStruct((2, NPAD, 128), jnp.float32) for _ in range(T)],
        scratch_types=[
            pltpu.VMEM((CPT // 2, CHUNK), jnp.int32),
            pltpu.VMEM((CPT // 2, CHUNK), jnp.int32),
            pltpu.VMEM((CHUNK, 128), jnp.float32),
            pltpu.VMEM((CHUNK, 128), jnp.float32),
            pltpu.VMEM_SHARED((NPAD, 128), jnp.float32),
            pltpu.SemaphoreType.DMA,
            pltpu.SemaphoreType.DMA,
        ],
    )
    def k(*refs):
        us = refs[:T]
        src_hbm, dst_hbm, z_hbm = refs[T:T + 3]
        outs = refs[T + 3:2 * T + 3]
        srcv, dstv, rows0, rows1, acc, sem0, sem1 = refs[2 * T + 3:]
        rows = (rows0, rows1)
        sems = (sem0, sem1)
        c = lax.axis_index("c")
        s = lax.axis_index("s")
        wid = c * 16 + s
        hcpt = CPT // 2

        for t in range(T):
            u = us[t]
            pltpu.sync_copy(z_hbm, rows0)
            for j in range(640 // CHUNK):
                pltpu.sync_copy(rows0, acc.at[pl.ds(s * 640 + j * CHUNK, CHUNK)])
            plsc.subcore_barrier()

            for half in range(2):
                base = wid * CPT + half * hcpt
                pltpu.sync_copy(src_hbm.at[pl.ds(base, hcpt)], srcv)
                pltpu.sync_copy(dst_hbm.at[pl.ds(base, hcpt)], dstv)

                for b in range(2):
                    pltpu.make_async_copy(u.at[srcv.at[b]], rows[b], sems[b]).start()

                def body(i, _):
                    for b in range(2):
                        g = 2 * i + b
                        pltpu.make_async_copy(u.at[srcv.at[g]], rows[b], sems[b]).wait()
                        pltpu.sync_copy(rows[b], acc.at[dstv.at[g]], add=True)
                        nxt = g + 2

                        @pl.when(nxt < hcpt)
                        def _():
                            pltpu.make_async_copy(
                                u.at[srcv.at[nxt]], rows[b], sems[b]).start()

                    return 0

                lax.fori_loop(0, hcpt // 2, body, 0)
            plsc.subcore_barrier()
            pltpu.sync_copy(acc.at[pl.ds(s * 640, 640)],
                            outs[t].at[c, pl.ds(s * 640, 640)])
            if t + 1 < T:
                plsc.subcore_barrier()

    out = k(*tables, src2d, dst2d, zeros128)
    return list(out) if isinstance(out, (list, tuple)) else [out]


def _dot(a, b):
    return jnp.dot(a, b, preferred_element_type=jnp.float32)


def _k_dinv(degp):
    def body(d_ref, o_ref):
        s = d_ref[0, :, 0:1] + d_ref[1, :, 0:1] + 1.0
        o_ref[...] = 1.0 / jnp.sqrt(s)

    return pl.pallas_call(
        body,
        grid=(10,),
        in_specs=[pl.BlockSpec((2, 1000, 128), lambda g: (0, g, 0))],
        out_specs=pl.BlockSpec((1000, 1), lambda g: (g, 0)),
        out_shape=jax.ShapeDtypeStruct((N, 1), jnp.float32),
    )(degp)


def _k_embed(ev, cs, tr, env, act, dinv, p1, p2):
    def body(ev_ref, cs_ref, tr_ref, env_ref, act_ref, dinv_ref,
             wev1, bev1, wcs1, bcs1, wtr1, btr1, wenv1, benv1, wg0x1, wg0a1,
             wev2, bev2, wcs2, bcs2, wtr2, btr2, wenv2, benv2, wg0x2, wg0a2,
             u1_ref, u2_ref):
        g = pl.program_id(0)
        a = act_ref[...]
        di = dinv_ref[...]
        branches = [
            (0, 4, ev_ref, wev1, bev1, wev2, bev2),
            (4, 7, cs_ref, wcs1, bcs1, wcs2, bcs2),
            (7, 9, tr_ref, wtr1, btr1, wtr2, btr2),
            (9, 10, env_ref, wenv1, benv1, wenv2, benv2),
        ]
        for lo, hi, src_ref, wA, bA, wB, bB in branches:
            @pl.when((g >= lo) & (g < hi))
            def _(lo=lo, src_ref=src_ref, wA=wA, bA=bA, wB=wB, bB=bB):
                blk = src_ref[pl.ds((g - lo) * 1000, 1000), :]
                x1 = jax.nn.relu(_dot(blk, wA[...]) + bA[...])
                x2 = jax.nn.relu(_dot(blk, wB[...]) + bB[...])
                u1_ref[...] = di * (_dot(x1, wg0x1[...]) + a * wg0a1[...])
                u2_ref[...] = di * (_dot(x2, wg0x2[...]) + a * wg0a2[...])

    full = lambda shape: pl.BlockSpec(shape, lambda g: tuple(0 for _ in shape))
    rowblk = pl.BlockSpec((1000, 1), lambda g: (g, 0))
    w1 = [full(p1[k].shape) for k in ("W_ev", "b_ev", "W_cs", "b_cs", "W_tr",
                                      "b_tr", "W_env", "b_env", "W_g0x", "W_g0a")]
    w2 = [full(p2[k].shape) for k in ("W_ev", "b_ev", "W_cs", "b_cs", "W_tr",
                                      "b_tr", "W_env", "b_env", "W_g0x", "W_g0a")]
    args1 = [p1[k] for k in ("W_ev", "b_ev", "W_cs", "b_cs", "W_tr", "b_tr",
                             "W_env", "b_env", "W_g0x", "W_g0a")]
    args2 = [p2[k] for k in ("W_ev", "b_ev", "W_cs", "b_cs", "W_tr", "b_tr",
                             "W_env", "b_env", "W_g0x", "W_g0a")]
    return pl.pallas_call(
        body,
        grid=(10,),
        in_specs=[full(ev.shape), full(cs.shape), full(tr.shape), full(env.shape),
                  rowblk, rowblk] + w1 + w2,
        out_specs=[pl.BlockSpec((1000, 128), lambda g: (g, 0))] * 2,
        out_shape=[jax.ShapeDtypeStruct((N, 128), jnp.float32)] * 2,
    )(ev, cs, tr, env, act, dinv, *args1, *args2)


def _k_mid(r0_1, r0_2, u0_1, u0_2, dinv, bg0_1, bg0_2):
    def body(r1_ref, r2_ref, u1_ref, u2_ref, dinv_ref, b1_ref, b2_ref,
             o1_ref, o2_ref):
        di = dinv_ref[...]
        h1 = jax.nn.relu(di * (r1_ref[0] + r1_ref[1] + u1_ref[...]) + b1_ref[...])
        h2 = jax.nn.relu(di * (r2_ref[0] + r2_ref[1] + u2_ref[...]) + b2_ref[...])
        o1_ref[...] = di * h1
        o2_ref[...] = di * h2

    rblk = pl.BlockSpec((2, 1000, 128), lambda g: (0, g, 0))
    ublk = pl.BlockSpec((1000, 128), lambda g: (g, 0))
    bblk = pl.BlockSpec((1, 128), lambda g: (0, 0))
    return pl.pallas_call(
        body,
        grid=(10,),
        in_specs=[rblk, rblk, ublk, ublk, pl.BlockSpec((1000, 1), lambda g: (g, 0)),
                  bblk, bblk],
        out_specs=[ublk, ublk],
        out_shape=[jax.ShapeDtypeStruct((N, 128), jnp.float32)] * 2,
    )(r0_1, r0_2, u0_1, u0_2, dinv, bg0_1, bg0_2)


def _k_l2(r1_1, r1_2, u1_1, u1_2, dinv, wg1_1, bg1_1, wg1_2, bg1_2):
    def body(r1_ref, r2_ref, u1_ref, u2_ref, dinv_ref, w1_ref, b1_ref,
             w2_ref, b2_ref, o1a, o1b, o2a, o2b):
        di = dinv_ref[...]
        agg1 = di * (r1_ref[0] + r1_ref[1] + u1_ref[...])
        agg2 = di * (r2_ref[0] + r2_ref[1] + u2_ref[...])
        h1 = di * jax.nn.relu(_dot(agg1, w1_ref[...]) + b1_ref[...])
        h2 = di * jax.nn.relu(_dot(agg2, w2_ref[...]) + b2_ref[...])
        o1a[...] = h1[:, :128]
        o1b[...] = h1[:, 128:]
        o2a[...] = h2[:, :128]
        o2b[...] = h2[:, 128:]

    rblk = pl.BlockSpec((2, 1000, 128), lambda g: (0, g, 0))
    ublk = pl.BlockSpec((1000, 128), lambda g: (g, 0))
    return pl.pallas_call(
        body,
        grid=(10,),
        in_specs=[rblk, rblk, ublk, ublk, pl.BlockSpec((1000, 1), lambda g: (g, 0)),
                  pl.BlockSpec((128, 256), lambda g: (0, 0)),
                  pl.BlockSpec((1, 256), lambda g: (0, 0)),
                  pl.BlockSpec((128, 256), lambda g: (0, 0)),
                  pl.BlockSpec((1, 256), lambda g: (0, 0))],
        out_specs=[ublk] * 4,
        out_shape=[jax.ShapeDtypeStruct((N, 128), jnp.float32)] * 4,
    )(r1_1, r1_2, u1_1, u1_2, dinv, wg1_1, bg1_1, wg1_2, bg1_2)


def _k_final(r2s, u2s, dinv, p1, p2):
    def body(r1a, r1b, r2a, r2b, u1a, u1b, u2a, u2b, dinv_ref,
             wg2_1, bg2_1, w1_1, b1_1, w2_1, b2_1, w3_1, b3_1,
             wg2_2, bg2_2, w1_2, b1_2, w2_2, b2_2, w3_2, b3_2,
             q1_ref, q2_ref):
        di = dinv_ref[...]

        def critic(ra, rb, ua, ub, wg2, bg2, w1, b1, w2, b2, w3, b3, q_ref):
            agga = di * (ra[0] + ra[1] + ua[...])
            aggb = di * (rb[0] + rb[1] + ub[...])
            h2 = jax.nn.relu(_dot(agga, wg2[pl.ds(0, 128), :])
                             + _dot(aggb, wg2[pl.ds(128, 128), :]) + bg2[...])
            pooled = jnp.sum(h2, axis=0, keepdims=True) * (1.0 / 1000.0)
            y = jax.nn.relu(_dot(pooled, w1[...]) + b1[...])
            y = jax.nn.relu(_dot(y, w2[...]) + b2[...])
            q = _dot(y, w3[...]) + b3[...]
            q_ref[...] = jnp.broadcast_to(q, (1, 1, 128))

        critic(r1a, r1b, u1a, u1b, wg2_1, bg2_1, w1_1, b1_1, w2_1, b2_1,
               w3_1, b3_1, q1_ref)
        critic(r2a, r2b, u2a, u2b, wg2_2, bg2_2, w1_2, b1_2, w2_2, b2_2,
               w3_2, b3_2, q2_ref)

    rblk = pl.BlockSpec((2, 1000, 128), lambda g: (0, g, 0))
    ublk = pl.BlockSpec((1000, 128), lambda g: (g, 0))
    full = lambda shape: pl.BlockSpec(shape, lambda g: tuple(0 for _ in shape))
    wkeys = ("W_g2", "b_g2", "W1", "b1", "W2", "b2", "W3", "b3")
    wspecs = [full(p1[k].shape) for k in wkeys] + [full(p2[k].shape) for k in wkeys]
    wargs = [p1[k] for k in wkeys] + [p2[k] for k in wkeys]
    q1, q2 = pl.pallas_call(
        body,
        grid=(10,),
        in_specs=[rblk] * 4 + [ublk] * 4
                 + [pl.BlockSpec((1000, 1), lambda g: (g, 0))] + wspecs,
        out_specs=[pl.BlockSpec((1, 1, 128), lambda g: (g, 0, 0))] * 2,
        out_shape=[jax.ShapeDtypeStruct((10, 1, 128), jnp.float32)] * 2,
    )(*r2s, *u2s, dinv, *wargs)
    return q1[:, 0, :1], q2[:, 0, :1]


def _prep_params(p):
    q = dict(p)
    for k in ("b_ev", "b_cs", "b_tr", "b_env", "b_g0", "b_g1", "b_g2",
              "b1", "b2", "b3"):
        q[k] = p[k].reshape(1, -1)
    q["W_g0x"] = p["W_g0"][:128]
    q["W_g0a"] = p["W_g0"][128].reshape(1, -1)
    return q


def kernel(ev_features, cs_features, tr_features, env_features, edge_index,
           ev_indexes, cs_indexes, tr_indexes, env_indexes, action, batch,
           params1, params2):
    src = edge_index[0]
    dst = edge_index[1]
    npad = EPAD - E
    src2d = jnp.concatenate([src, jnp.zeros((npad,), jnp.int32)]).reshape(-1, CHUNK)
    dst2d = jnp.concatenate([dst, jnp.full((npad,), N, jnp.int32)]).reshape(-1, CHUNK)
    zeros128 = jnp.zeros((CHUNK, 128), jnp.float32)
    ones128 = jnp.ones((CHUNK, 128), jnp.float32)
    act = action.reshape(-1, 1)
    p1 = _prep_params(params1)
    p2 = _prep_params(params2)

    degp = _sc_deg(dst2d, zeros128, ones128)
    dinv = _k_dinv(degp)
    u0_1, u0_2 = _k_embed(ev_features, cs_features, tr_features, env_features,
                          act, dinv, p1, p2)
    r0_1, r0_2 = _sc_agg([u0_1, u0_2], src2d, dst2d, zeros128)
    u1_1, u1_2 = _k_mid(r0_1, r0_2, u0_1, u0_2, dinv, p1["b_g0"], p2["b_g0"])
    r1_1, r1_2 = _sc_agg([u1_1, u1_2], src2d, dst2d, zeros128)
    u2 = _k_l2(r1_1, r1_2, u1_1, u1_2, dinv, p1["W_g1"], p1["b_g1"],
               p2["W_g1"], p2["b_g1"])
    r2 = _sc_agg(list(u2), src2d, dst2d, zeros128)
    return _k_final(r2, u2, dinv, p1, p2)
